# baseline (speedup 1.0000x reference)
import numpy as np
import jax
import jax.numpy as jnp
from jax.experimental import pallas as pl
from jax.experimental.pallas import tpu as pltpu

CLAMP = 10.0
BBLK = 512
CHUNK = 128


def _made_masks(D, H):
    m_in = np.arange(D)
    m0 = np.arange(H) % (D - 1)
    mask1 = (m_in[None, :] <= m0[:, None]).astype(np.float32)
    base = (m0[None, :] < m_in[:, None]).astype(np.float32)
    mask2 = np.repeat(base, 2, axis=0).astype(np.float32)
    return mask1, mask2, m0


def _plan(D, H, sorted_m0):
    nch = H // CHUNK
    cmin = [int(sorted_m0[c * CHUNK]) for c in range(nch)]
    cmax = [int(sorted_m0[(c + 1) * CHUNK - 1]) for c in range(nch)]
    steps = []
    for i in range(D):
        d_al = (D + i) // 2
        d_mu = i // 2
        ac = [c for c in range(nch) if cmin[c] < d_al]
        mc = set(c for c in range(nch) if cmin[c] < d_mu)
        uc = [c for c in range(nch) if cmax[c] >= i]
        steps.append((ac, mc, uc))
    return nch, steps


def _make_body(D, H, steps, nch):
    def body(z_ref, w1t_ref, b1_ref, wmu_ref, wal_ref, b2mu_ref, b2al_ref,
             x_ref, ld_ref, acc_ref, t_ref):
        Bb = z_ref.shape[0]
        acc_ref[...] = jnp.broadcast_to(b1_ref[...], (Bb, H))
        z = z_ref[...]
        ld = jnp.zeros((Bb, 1), jnp.float32)
        valid = [False] * nch
        cols = []
        for i in range(D):
            ac, mc, uc = steps[i]
            for c in ac:
                if not valid[c]:
                    sl = slice(c * CHUNK, (c + 1) * CHUNK)
                    t_ref[:, sl] = jnp.tanh(acc_ref[:, sl])
                    valid[c] = True
            mu_p = None
            al_p = None
            for c in ac:
                sl = slice(c * CHUNK, (c + 1) * CHUNK)
                tc = t_ref[:, sl]
                ca = tc * wal_ref[i:i + 1, sl]
                al_p = ca if al_p is None else al_p + ca
                if c in mc:
                    cm = tc * wmu_ref[i:i + 1, sl]
                    mu_p = cm if mu_p is None else mu_p + cm
            zero = jnp.zeros((Bb, 1), jnp.float32)
            al_raw = (jnp.sum(al_p, axis=1, keepdims=True)
                      if al_p is not None else zero)
            mu_raw = (jnp.sum(mu_p, axis=1, keepdims=True)
                      if mu_p is not None else zero)
            mu = jnp.clip(mu_raw + b2mu_ref[0:1, i:i + 1], -CLAMP, CLAMP)
            al = jnp.clip(al_raw + b2al_ref[0:1, i:i + 1], -CLAMP, CLAMP)
            x_i = z[:, i:i + 1] * jnp.exp(al) + mu
            for c in uc:
                sl = slice(c * CHUNK, (c + 1) * CHUNK)
                acc_ref[:, sl] = acc_ref[:, sl] + x_i * w1t_ref[i:i + 1, sl]
                valid[c] = False
            cols.append(x_i)
            ld = ld + al
        x = jnp.concatenate(cols, axis=1)
        x_ref[...] = jnp.where(jnp.isnan(x) | jnp.isinf(x), 0.0, x)
        ld_ref[...] = jnp.where(jnp.isnan(ld) | jnp.isinf(ld), 0.0, ld)
    return body


def kernel(z, W1, b1, W2, b2):
    B, D = z.shape
    H = W1.shape[0]
    mask1, mask2, m0 = _made_masks(D, H)
    perm = np.argsort(m0, kind="stable")
    sorted_m0 = m0[perm]
    nch, steps = _plan(D, H, sorted_m0)

    w1t = (W1 * mask1).T[:, perm]
    W2m = W2 * mask2
    wmu = W2m[:D, perm]
    wal = W2m[D:, perm]
    b1r = b1[perm].reshape(1, H)
    b2mu = b2[:D].reshape(1, D)
    b2al = b2[D:].reshape(1, D)

    x, ld = pl.pallas_call(
        _make_body(D, H, steps, nch),
        grid=(B // BBLK,),
        in_specs=[
            pl.BlockSpec((BBLK, D), lambda i: (i, 0)),
            pl.BlockSpec((D, H), lambda i: (0, 0)),
            pl.BlockSpec((1, H), lambda i: (0, 0)),
            pl.BlockSpec((D, H), lambda i: (0, 0)),
            pl.BlockSpec((D, H), lambda i: (0, 0)),
            pl.BlockSpec((1, D), lambda i: (0, 0)),
            pl.BlockSpec((1, D), lambda i: (0, 0)),
        ],
        out_specs=[
            pl.BlockSpec((BBLK, D), lambda i: (i, 0)),
            pl.BlockSpec((BBLK, 1), lambda i: (i, 0)),
        ],
        out_shape=[
            jax.ShapeDtypeStruct((B, D), jnp.float32),
            jax.ShapeDtypeStruct((B, 1), jnp.float32),
        ],
        scratch_shapes=[
            pltpu.VMEM((BBLK, H), jnp.float32),
            pltpu.VMEM((BBLK, H), jnp.float32),
        ],
        compiler_params=pltpu.CompilerParams(
            dimension_semantics=("parallel",),
        ),
    )(z, w1t, b1r, wmu, wal, b2mu, b2al)
    return x, ld.reshape(B)
